# baseline (speedup 1.0000x reference)
import functools

import jax
import jax.numpy as jnp
from jax.experimental import pallas as pl
from jax.experimental.pallas import tpu as pltpu

_GC = 0.7978845608028654
_GA = _GC * 0.044715


def _gelu2x(z):
    z2 = z * z
    q = z * (_GA * z2 + _GC)
    return z * (jnp.tanh(q) + 1.0)


def _leff_fused(x_ref, w1_ref, b1_ref, dwk_ref, dwb_ref, w2_ref, b2_ref,
                o_ref, slab_ref, *, hh, halo, chunk):
    bt, HW, _ = x_ref.shape
    hidden = w1_ref.shape[1]
    nchunks = HW // chunk

    col = jax.lax.broadcasted_iota(jnp.int32, (chunk, 1), 0) % hh
    not_left = (col != 0)
    not_right = (col != hh - 1)

    for b in range(bt):
        slab_ref[b, pl.ds(0, halo), :] = jnp.zeros((halo, hidden), jnp.float32)
        slab_ref[b, pl.ds(halo + HW, halo), :] = (
            jnp.zeros((halo, hidden), jnp.float32))

        def pass_a(c, carry):
            q0 = pl.multiple_of(c * chunk, 8)
            xc = x_ref[b, pl.ds(q0, chunk), :]
            z = jnp.dot(xc, w1_ref[...],
                        preferred_element_type=jnp.float32) + b1_ref[...]
            slab_ref[b, pl.ds(pl.multiple_of(halo, 8) + q0, chunk), :] = (
                _gelu2x(z))
            return carry
        jax.lax.fori_loop(0, nchunks, pass_a, 0)

        def pass_b(c, carry):
            q0 = pl.multiple_of(c * chunk, 8)
            accL = jnp.zeros((chunk, hidden), jnp.float32)
            accC = jnp.zeros((chunk, hidden), jnp.float32)
            accR = jnp.zeros((chunk, hidden), jnp.float32)
            for dy in range(3):
                base = halo + (dy - 1) * hh
                accL += slab_ref[b, pl.ds(base - 1 + q0, chunk), :] * (
                    dwk_ref[3 * dy + 0:3 * dy + 1, :])
                accC += slab_ref[b, pl.ds(pl.multiple_of(base, 8) + q0,
                                          chunk), :] * (
                    dwk_ref[3 * dy + 1:3 * dy + 2, :])
                accR += slab_ref[b, pl.ds(base + 1 + q0, chunk), :] * (
                    dwk_ref[3 * dy + 2:3 * dy + 3, :])
            acc = accC + jnp.where(not_left, accL, 0.0)
            acc = acc + jnp.where(not_right, accR, 0.0)
            h2 = _gelu2x(acc + dwb_ref[...])
            y = jnp.dot(h2, w2_ref[...],
                        preferred_element_type=jnp.float32) + b2_ref[...]
            o_ref[b, pl.ds(q0, chunk), :] = y.astype(o_ref.dtype)
            return carry
        jax.lax.fori_loop(0, nchunks, pass_b, 0)


def kernel(x, w1, b1, dw_w, dw_b, w2, b2, *, block_b=4, chunk=128):
    B, HW, dim = x.shape
    hh = int(round(HW ** 0.5))
    assert hh * hh == HW, "token count must be a perfect square"
    hidden = w1.shape[1]

    if chunk > HW or HW % chunk != 0 or chunk % hh != 0:
        chunk = HW
    halo = -(-(hh + 1) // 8) * 8
    R = 2 * halo + HW

    block_b = max(1, min(block_b, B))
    Bp = -(-B // block_b) * block_b
    xp = jnp.pad(x, ((0, Bp - B), (0, 0), (0, 0))) if Bp != B else x

    b1r = b1.reshape(1, hidden)
    dwbr = dw_b.reshape(1, hidden)
    dwk = 0.5 * dw_w.reshape(hidden, 9).T
    w2h = 0.5 * w2
    b2r = b2.reshape(1, dim)

    kfn = functools.partial(_leff_fused, hh=hh, halo=halo, chunk=chunk)

    est_bytes = 4 * (2 * block_b * HW * (dim + dim)
                     + block_b * R * hidden
                     + 2 * (dim * hidden + hidden * dim + 12 * hidden + dim))
    vmem_limit = int(min(max(2 * est_bytes, 32 * 1024 * 1024),
                         64 * 1024 * 1024))

    cost = pl.CostEstimate(
        flops=2 * B * HW * hidden * (2 * dim) + 18 * B * HW * hidden,
        transcendentals=2 * B * HW * hidden,
        bytes_accessed=4 * (Bp * HW * 2 * dim + dim * hidden
                            + hidden * dim + 12 * hidden + dim),
    )

    out = pl.pallas_call(
        kfn,
        out_shape=jax.ShapeDtypeStruct((Bp, HW, dim), x.dtype),
        grid_spec=pltpu.PrefetchScalarGridSpec(
            num_scalar_prefetch=0,
            grid=(Bp // block_b,),
            in_specs=[
                pl.BlockSpec((block_b, HW, dim), lambda g: (g, 0, 0)),
                pl.BlockSpec((dim, hidden), lambda g: (0, 0)),
                pl.BlockSpec((1, hidden), lambda g: (0, 0)),
                pl.BlockSpec((9, hidden), lambda g: (0, 0)),
                pl.BlockSpec((1, hidden), lambda g: (0, 0)),
                pl.BlockSpec((hidden, dim), lambda g: (0, 0)),
                pl.BlockSpec((1, dim), lambda g: (0, 0)),
            ],
            out_specs=pl.BlockSpec((block_b, HW, dim), lambda g: (g, 0, 0)),
            scratch_shapes=[
                pltpu.VMEM((block_b, R, hidden), jnp.float32),
            ],
        ),
        compiler_params=pltpu.CompilerParams(
            dimension_semantics=("parallel",),
            vmem_limit_bytes=vmem_limit),
        cost_estimate=cost,
    )(xp, w1, b1r, dwk, dwbr, w2h, b2r)

    return out[:B] if Bp != B else out


# speedup vs baseline: 2.3439x; 2.3439x over previous
import functools

import jax
import jax.numpy as jnp
from jax.experimental import pallas as pl
from jax.experimental.pallas import tpu as pltpu

_GC = 0.7978845608028654
_GA = _GC * 0.044715


def _gelu2x(z):
    z2 = z * z
    q = z * (_GA * z2 + _GC)
    return z * (jnp.tanh(q) + 1.0)


def _leff_fused(x_ref, w1_ref, b1_ref, dwk_ref, dwb_ref, w2_ref, b2_ref,
                o_ref, slab_ref, *, hh, halo, chunk):
    bt, HW, _ = x_ref.shape
    hidden = w1_ref.shape[1]
    nchunks = HW // chunk

    col = jax.lax.broadcasted_iota(jnp.int32, (chunk, 1), 0) % hh
    not_left = (col != 0)
    not_right = (col != hh - 1)

    for b in range(bt):
        slab_ref[b, pl.ds(0, halo), :] = jnp.zeros((halo, hidden), jnp.float32)
        slab_ref[b, pl.ds(halo + HW, halo), :] = (
            jnp.zeros((halo, hidden), jnp.float32))

        for c in range(nchunks):
            q0 = c * chunk
            xc = x_ref[b, pl.ds(q0, chunk), :]
            z = jnp.dot(xc, w1_ref[...],
                        preferred_element_type=jnp.float32) + b1_ref[...]
            slab_ref[b, pl.ds(halo + q0, chunk), :] = _gelu2x(z)

        for c in range(nchunks):
            q0 = c * chunk
            accL = jnp.zeros((chunk, hidden), jnp.float32)
            accC = jnp.zeros((chunk, hidden), jnp.float32)
            accR = jnp.zeros((chunk, hidden), jnp.float32)
            for dy in range(3):
                base = halo + (dy - 1) * hh + q0
                accL += slab_ref[b, pl.ds(base - 1, chunk), :] * (
                    dwk_ref[3 * dy + 0:3 * dy + 1, :])
                accC += slab_ref[b, pl.ds(pl.multiple_of(base, 8), chunk), :] * (
                    dwk_ref[3 * dy + 1:3 * dy + 2, :])
                accR += slab_ref[b, pl.ds(base + 1, chunk), :] * (
                    dwk_ref[3 * dy + 2:3 * dy + 3, :])
            acc = accC + jnp.where(not_left, accL, 0.0)
            acc = acc + jnp.where(not_right, accR, 0.0)
            h2 = _gelu2x(acc + dwb_ref[...])
            y = jnp.dot(h2, w2_ref[...],
                        preferred_element_type=jnp.float32) + b2_ref[...]
            o_ref[b, pl.ds(q0, chunk), :] = y.astype(o_ref.dtype)


def kernel(x, w1, b1, dw_w, dw_b, w2, b2, *, block_b=4, chunk=128):
    B, HW, dim = x.shape
    hh = int(round(HW ** 0.5))
    assert hh * hh == HW, "token count must be a perfect square"
    hidden = w1.shape[1]

    if chunk > HW or HW % chunk != 0 or chunk % hh != 0:
        chunk = HW
    halo = -(-(hh + 1) // 8) * 8
    R = 2 * halo + HW

    block_b = max(1, min(block_b, B))
    Bp = -(-B // block_b) * block_b
    xp = jnp.pad(x, ((0, Bp - B), (0, 0), (0, 0))) if Bp != B else x

    b1r = b1.reshape(1, hidden)
    dwbr = dw_b.reshape(1, hidden)
    dwk = 0.5 * dw_w.reshape(hidden, 9).T
    w2h = 0.5 * w2
    b2r = b2.reshape(1, dim)

    kfn = functools.partial(_leff_fused, hh=hh, halo=halo, chunk=chunk)

    est_bytes = 4 * (2 * block_b * HW * (dim + dim)
                     + block_b * R * hidden
                     + 2 * (dim * hidden + hidden * dim + 12 * hidden + dim))
    vmem_limit = int(min(max(2 * est_bytes, 32 * 1024 * 1024),
                         64 * 1024 * 1024))

    cost = pl.CostEstimate(
        flops=2 * B * HW * hidden * (2 * dim) + 18 * B * HW * hidden,
        transcendentals=2 * B * HW * hidden,
        bytes_accessed=4 * (Bp * HW * 2 * dim + dim * hidden
                            + hidden * dim + 12 * hidden + dim),
    )

    out = pl.pallas_call(
        kfn,
        out_shape=jax.ShapeDtypeStruct((Bp, HW, dim), x.dtype),
        grid_spec=pltpu.PrefetchScalarGridSpec(
            num_scalar_prefetch=0,
            grid=(Bp // block_b,),
            in_specs=[
                pl.BlockSpec((block_b, HW, dim), lambda g: (g, 0, 0)),
                pl.BlockSpec((dim, hidden), lambda g: (0, 0)),
                pl.BlockSpec((1, hidden), lambda g: (0, 0)),
                pl.BlockSpec((9, hidden), lambda g: (0, 0)),
                pl.BlockSpec((1, hidden), lambda g: (0, 0)),
                pl.BlockSpec((hidden, dim), lambda g: (0, 0)),
                pl.BlockSpec((1, dim), lambda g: (0, 0)),
            ],
            out_specs=pl.BlockSpec((block_b, HW, dim), lambda g: (g, 0, 0)),
            scratch_shapes=[
                pltpu.VMEM((block_b, R, hidden), jnp.float32),
            ],
        ),
        compiler_params=pltpu.CompilerParams(
            dimension_semantics=("parallel",),
            vmem_limit_bytes=vmem_limit),
        cost_estimate=cost,
    )(xp, w1, b1r, dwk, dwbr, w2h, b2r)

    return out[:B] if Bp != B else out


# speedup vs baseline: 2.5597x; 1.0921x over previous
import functools

import jax
import jax.numpy as jnp
from jax.experimental import pallas as pl
from jax.experimental.pallas import tpu as pltpu

_GC = 0.7978845608028654
_GA = _GC * 0.044715


def _gelu2x(z):
    z2 = z * z
    q = z * (_GA * z2 + _GC)
    return z * (jnp.tanh(q) + 1.0)


def _leff_fused(x_ref, w1_ref, b1_ref, dwk_ref, dwb_ref, w2_ref, b2_ref,
                o_ref, slab_ref, *, hh, xpad):
    bt, HW, _ = x_ref.shape
    hidden = w1_ref.shape[1]
    X = slab_ref.shape[2]

    for b in range(bt):
        slab_ref[b, :, pl.ds(0, xpad), :] = (
            jnp.zeros((hh + 2, xpad, hidden), jnp.float32))
        slab_ref[b, :, pl.ds(xpad + hh, X - xpad - hh), :] = (
            jnp.zeros((hh + 2, X - xpad - hh, hidden), jnp.float32))
        slab_ref[b, 0, pl.ds(xpad, hh), :] = (
            jnp.zeros((hh, hidden), jnp.float32))
        slab_ref[b, hh + 1, pl.ds(xpad, hh), :] = (
            jnp.zeros((hh, hidden), jnp.float32))

        z = jnp.dot(x_ref[b], w1_ref[...],
                    preferred_element_type=jnp.float32) + b1_ref[...]
        h = _gelu2x(z).reshape(hh, hh, hidden)
        slab_ref[b, pl.ds(1, hh), pl.ds(xpad, hh), :] = h

        acc = None
        for dy in range(3):
            for dx in range(3):
                t = 3 * dy + dx
                v = slab_ref[b, pl.ds(dy, hh), pl.ds(xpad - 1 + dx, hh), :] * (
                    dwk_ref[t:t + 1, :])
                acc = v if acc is None else acc + v
        h2 = _gelu2x(acc + dwb_ref[...]).reshape(HW, hidden)
        y = jnp.dot(h2, w2_ref[...],
                    preferred_element_type=jnp.float32) + b2_ref[...]
        o_ref[b] = y.astype(o_ref.dtype)


def kernel(x, w1, b1, dw_w, dw_b, w2, b2, *, block_b=8):
    B, HW, dim = x.shape
    hh = int(round(HW ** 0.5))
    assert hh * hh == HW, "token count must be a perfect square"
    hidden = w1.shape[1]

    xpad = 8
    X = xpad + hh + xpad

    block_b = max(1, min(block_b, B))
    Bp = -(-B // block_b) * block_b
    xp = jnp.pad(x, ((0, Bp - B), (0, 0), (0, 0))) if Bp != B else x

    b1r = b1.reshape(1, hidden)
    dwbr = dw_b.reshape(1, hidden)
    dwk = 0.5 * dw_w.reshape(hidden, 9).T
    w2h = 0.5 * w2
    b2r = b2.reshape(1, dim)

    kfn = functools.partial(_leff_fused, hh=hh, xpad=xpad)

    est_bytes = 4 * (2 * block_b * HW * (dim + dim)
                     + block_b * (hh + 2) * X * hidden
                     + 2 * (dim * hidden + hidden * dim + 12 * hidden + dim))
    vmem_limit = int(min(max(2 * est_bytes, 32 * 1024 * 1024),
                         64 * 1024 * 1024))

    cost = pl.CostEstimate(
        flops=2 * B * HW * hidden * (2 * dim) + 18 * B * HW * hidden,
        transcendentals=2 * B * HW * hidden,
        bytes_accessed=4 * (Bp * HW * 2 * dim + dim * hidden
                            + hidden * dim + 12 * hidden + dim),
    )

    out = pl.pallas_call(
        kfn,
        out_shape=jax.ShapeDtypeStruct((Bp, HW, dim), x.dtype),
        grid_spec=pltpu.PrefetchScalarGridSpec(
            num_scalar_prefetch=0,
            grid=(Bp // block_b,),
            in_specs=[
                pl.BlockSpec((block_b, HW, dim), lambda g: (g, 0, 0)),
                pl.BlockSpec((dim, hidden), lambda g: (0, 0)),
                pl.BlockSpec((1, hidden), lambda g: (0, 0)),
                pl.BlockSpec((9, hidden), lambda g: (0, 0)),
                pl.BlockSpec((1, hidden), lambda g: (0, 0)),
                pl.BlockSpec((hidden, dim), lambda g: (0, 0)),
                pl.BlockSpec((1, dim), lambda g: (0, 0)),
            ],
            out_specs=pl.BlockSpec((block_b, HW, dim), lambda g: (g, 0, 0)),
            scratch_shapes=[
                pltpu.VMEM((block_b, hh + 2, X, hidden), jnp.float32),
            ],
        ),
        compiler_params=pltpu.CompilerParams(
            dimension_semantics=("parallel",),
            vmem_limit_bytes=vmem_limit),
        cost_estimate=cost,
    )(xp, w1, b1r, dwk, dwbr, w2h, b2r)

    return out[:B] if Bp != B else out
